# baseline (device time: 44099 ns/iter reference)
import jax
import jax.numpy as jnp
from jax import lax
from jax.experimental import pallas as pl
from jax.experimental.pallas import tpu as pltpu

N_DEV = 8
B, SQ, SKV, HL, DH = 2, 128, 128, 4, 64
DM = 512
DB = HL * DH


def kernel(x, Wq, K_ext, V_ext, Wo):
    idx = lax.axis_index("i")
    Wq_sl = lax.dynamic_slice_in_dim(Wq, idx * DB, DB, axis=1).astype(jnp.bfloat16)
    Wo_sl = lax.dynamic_slice_in_dim(Wo, idx * DB, DB, axis=0).astype(jnp.bfloat16)
    xf = x.reshape(B * SQ, DM).astype(jnp.bfloat16)
    Kf = K_ext.transpose(0, 2, 1, 3).reshape(B * HL, SKV, DH).astype(jnp.bfloat16)
    Vf = V_ext.transpose(0, 2, 1, 3).reshape(B * HL, SKV, DH).astype(jnp.bfloat16)

    def body(x_ref, wq_ref, k_ref, v_ref, wo_ref, out_ref,
             comm_ref, send_sems, recv_sems):
        my = lax.axis_index("i")
        left = lax.rem(my + N_DEV - 1, N_DEV)
        right = lax.rem(my + 1, N_DEV)

        barrier = pltpu.get_barrier_semaphore()
        for nbr in (left, right):
            pl.semaphore_signal(barrier, inc=1, device_id=(nbr,),
                                device_id_type=pl.DeviceIdType.MESH)
        pl.semaphore_wait(barrier, 2)

        q = lax.dot(x_ref[...], wq_ref[...],
                    preferred_element_type=jnp.float32).astype(jnp.bfloat16)
        ctx_rows = []
        for b in range(B):
            head_cols = []
            for h in range(HL):
                qbh = q[b * SQ:(b + 1) * SQ, h * DH:(h + 1) * DH]
                kbh = k_ref[b * HL + h]
                vbh = v_ref[b * HL + h]
                s = lax.dot_general(
                    qbh, kbh, (((1,), (1,)), ((), ())),
                    preferred_element_type=jnp.float32) * 0.125
                s = s - jnp.max(s, axis=1, keepdims=True)
                w = jnp.exp(s)
                w = w / jnp.sum(w, axis=1, keepdims=True)
                ctx = lax.dot(w.astype(jnp.bfloat16), vbh,
                              preferred_element_type=jnp.float32)
                head_cols.append(ctx.astype(jnp.bfloat16))
            ctx_rows.append(jnp.concatenate(head_cols, axis=1))
        ctx_all = jnp.concatenate(ctx_rows, axis=0)
        partial = lax.dot(ctx_all, wo_ref[...],
                          preferred_element_type=jnp.float32)

        comm_ref[0, :, :] = partial.astype(jnp.bfloat16)
        acc = partial
        for hop in range(N_DEV - 1):
            send_slot = hop % 2
            recv_slot = (hop + 1) % 2
            rdma = pltpu.make_async_remote_copy(
                src_ref=comm_ref.at[send_slot],
                dst_ref=comm_ref.at[recv_slot],
                send_sem=send_sems.at[send_slot],
                recv_sem=recv_sems.at[recv_slot],
                device_id=(right,),
                device_id_type=pl.DeviceIdType.MESH,
            )
            rdma.start()
            rdma.wait()
            acc = acc + comm_ref[recv_slot, :, :].astype(jnp.float32)
        out_ref[...] = acc.reshape(B, SQ, DM)

    return pl.pallas_call(
        body,
        out_shape=jax.ShapeDtypeStruct((B, SQ, DM), jnp.float32),
        in_specs=[pl.BlockSpec(memory_space=pltpu.VMEM)] * 5,
        out_specs=pl.BlockSpec(memory_space=pltpu.VMEM),
        scratch_shapes=[
            pltpu.VMEM((2, B * SQ, DM), jnp.bfloat16),
            pltpu.SemaphoreType.DMA((2,)),
            pltpu.SemaphoreType.DMA((2,)),
        ],
        compiler_params=pltpu.CompilerParams(collective_id=0),
    )(xf, Wq_sl, Kf, Vf, Wo_sl)


# device time: 25528 ns/iter; 1.7275x vs baseline; 1.7275x over previous
import jax
import jax.numpy as jnp
from jax import lax
from jax.experimental import pallas as pl
from jax.experimental.pallas import tpu as pltpu

N_DEV = 8
B, SQ, SKV, HL, DH = 2, 128, 128, 4, 64
DM = 512
DB = HL * DH


def kernel(x, Wq, K_ext, V_ext, Wo):
    idx = lax.axis_index("i")
    Wq_sl = lax.dynamic_slice_in_dim(Wq, idx * DB, DB, axis=1).astype(jnp.bfloat16)
    Wo_sl = lax.dynamic_slice_in_dim(Wo, idx * DB, DB, axis=0).astype(jnp.bfloat16)
    xf = x.reshape(B * SQ, DM).astype(jnp.bfloat16)
    Kf = K_ext.transpose(0, 2, 1, 3).reshape(B * HL, SKV, DH).astype(jnp.bfloat16)
    Vf = V_ext.transpose(0, 2, 1, 3).reshape(B * HL, SKV, DH).astype(jnp.bfloat16)

    def body(x_ref, wq_ref, k_ref, v_ref, wo_ref, out_ref,
             send_ref, recv_ref, send_sems, recv_sems):
        my = lax.axis_index("i")
        partners = [lax.bitwise_xor(my, c) for c in (1, 3, 4)]

        barrier = pltpu.get_barrier_semaphore()
        for nbr in partners:
            pl.semaphore_signal(barrier, inc=1, device_id=(nbr,),
                                device_id_type=pl.DeviceIdType.MESH)
        pl.semaphore_wait(barrier, len(partners))

        q = lax.dot(x_ref[...], wq_ref[...],
                    preferred_element_type=jnp.float32).astype(jnp.bfloat16)
        ctx_rows = []
        for b in range(B):
            head_cols = []
            for h in range(HL):
                qbh = q[b * SQ:(b + 1) * SQ, h * DH:(h + 1) * DH]
                kbh = k_ref[b * HL + h]
                vbh = v_ref[b * HL + h]
                s = lax.dot_general(
                    qbh, kbh, (((1,), (1,)), ((), ())),
                    preferred_element_type=jnp.float32) * 0.125
                s = s - jnp.max(s, axis=1, keepdims=True)
                w = jnp.exp(s)
                w = w / jnp.sum(w, axis=1, keepdims=True)
                ctx = lax.dot(w.astype(jnp.bfloat16), vbh,
                              preferred_element_type=jnp.float32)
                head_cols.append(ctx.astype(jnp.bfloat16))
            ctx_rows.append(jnp.concatenate(head_cols, axis=1))
        ctx_all = jnp.concatenate(ctx_rows, axis=0)
        partial = lax.dot(ctx_all, wo_ref[...],
                          preferred_element_type=jnp.float32)

        acc = partial
        for s, partner in enumerate(partners):
            send_ref[s, :, :] = acc.astype(jnp.bfloat16)
            rdma = pltpu.make_async_remote_copy(
                src_ref=send_ref.at[s],
                dst_ref=recv_ref.at[s],
                send_sem=send_sems.at[s],
                recv_sem=recv_sems.at[s],
                device_id=(partner,),
                device_id_type=pl.DeviceIdType.MESH,
            )
            rdma.start()
            rdma.wait()
            acc = acc + recv_ref[s, :, :].astype(jnp.float32)
        out_ref[...] = acc.reshape(B, SQ, DM)

    return pl.pallas_call(
        body,
        out_shape=jax.ShapeDtypeStruct((B, SQ, DM), jnp.float32),
        in_specs=[pl.BlockSpec(memory_space=pltpu.VMEM)] * 5,
        out_specs=pl.BlockSpec(memory_space=pltpu.VMEM),
        scratch_shapes=[
            pltpu.VMEM((3, B * SQ, DM), jnp.bfloat16),
            pltpu.VMEM((3, B * SQ, DM), jnp.bfloat16),
            pltpu.SemaphoreType.DMA((3,)),
            pltpu.SemaphoreType.DMA((3,)),
        ],
        compiler_params=pltpu.CompilerParams(collective_id=0),
    )(xf, Wq_sl, Kf, Vf, Wo_sl)
